# Initial kernel scaffold; baseline (speedup 1.0000x reference)
import jax
import jax.numpy as jnp
from jax.experimental import pallas as pl


def kernel(center_nodes, context_nodes, negative_nodes, u_weight, v_weight):
    raise NotImplementedError("write your pallas kernel here")


# speedup vs baseline: 1.7315x; 1.7315x over previous
import functools

import jax
import jax.numpy as jnp
from jax import lax
from jax.experimental import pallas as pl
from jax.experimental.pallas import tpu as pltpu
from jax.experimental.pallas import tpu_sc as plsc

B = 16384
D = 64
K = 5
NC = 2
NS = 16
NW = NC * NS
N_PER_W = B // NW
CHUNK = 128
N_CHUNKS = N_PER_W // CHUNK
L = 16
GROUPS = CHUNK // L


def _make_sc_scores():
    mesh = plsc.VectorSubcoreMesh(core_axis_name="c", subcore_axis_name="s")
    scratch = (
        [pltpu.VMEM((CHUNK,), jnp.int32) for _ in range(2 + K)]
        + [pltpu.VMEM((CHUNK, D), jnp.float32) for _ in range(2 + K)]
        + [pltpu.VMEM((N_PER_W,), jnp.float32) for _ in range(1 + K)]
        + [pltpu.SemaphoreType.DMA]
    )

    @functools.partial(
        pl.kernel,
        out_type=[
            jax.ShapeDtypeStruct((B,), jnp.float32),
            jax.ShapeDtypeStruct((K * B,), jnp.float32),
        ],
        mesh=mesh,
        scratch_types=scratch,
        compiler_params=pltpu.CompilerParams(
            needs_layout_passes=False, use_tc_tiling_on_sc=False),
    )
    def sc_scores(center_hbm, context_hbm, neg_hbm, u_hbm, v_hbm,
                  pos_out, neg_out,
                  idx_c, idx_x, idx_n0, idx_n1, idx_n2, idx_n3, idx_n4,
                  c_rows, x_rows, n0, n1, n2, n3, n4,
                  pos_v, nv0, nv1, nv2, nv3, nv4, sem):
        idx_n = [idx_n0, idx_n1, idx_n2, idx_n3, idx_n4]
        n_rows = [n0, n1, n2, n3, n4]
        neg_v = [nv0, nv1, nv2, nv3, nv4]
        wid = lax.axis_index("s") * NC + lax.axis_index("c")
        base_w = wid * N_PER_W

        for ci in range(N_CHUNKS):
            base = base_w + ci * CHUNK
            pltpu.sync_copy(center_hbm.at[pl.ds(base, CHUNK)], idx_c)
            pltpu.sync_copy(context_hbm.at[pl.ds(base, CHUNK)], idx_x)
            for kk in range(K):
                pltpu.sync_copy(neg_hbm.at[pl.ds(kk * B + base, CHUNK)], idx_n[kk])
            copies = [
                pltpu.async_copy(u_hbm.at[idx_c], c_rows, sem),
                pltpu.async_copy(v_hbm.at[idx_x], x_rows, sem),
            ]
            for kk in range(K):
                copies.append(pltpu.async_copy(v_hbm.at[idx_n[kk]], n_rows[kk], sem))
            for cp in copies:
                cp.wait()

            lanes = lax.iota(jnp.int32, L)
            for g in range(GROUPS):

                def elem_body(i, scores, g=g):
                    e = g * L + i
                    sel = lanes == i
                    c4 = [c_rows[e, pl.ds(q * L, L)] for q in range(D // L)]
                    x4 = [x_rows[e, pl.ds(q * L, L)] for q in range(D // L)]
                    acc = c4[0] * x4[0]
                    for q in range(1, D // L):
                        acc = acc + c4[q] * x4[q]
                    out = [jnp.where(sel, jnp.sum(acc), scores[0])]
                    for kk in range(K):
                        n4 = [n_rows[kk][e, pl.ds(q * L, L)] for q in range(D // L)]
                        acc = c4[0] * n4[0]
                        for q in range(1, D // L):
                            acc = acc + c4[q] * n4[q]
                        out.append(jnp.where(sel, jnp.sum(acc), scores[1 + kk]))
                    return tuple(out)

                init = tuple(jnp.zeros((L,), jnp.float32) for _ in range(1 + K))
                scores = lax.fori_loop(0, L, elem_body, init)
                off = ci * CHUNK + g * L
                pos_v[pl.ds(off, L)] = scores[0]
                for kk in range(K):
                    neg_v[kk][pl.ds(off, L)] = scores[1 + kk]

        pltpu.sync_copy(pos_v, pos_out.at[pl.ds(base_w, N_PER_W)])
        for kk in range(K):
            pltpu.sync_copy(neg_v[kk], neg_out.at[pl.ds(kk * B + base_w, N_PER_W)])

    return sc_scores


_sc_scores = _make_sc_scores()


def _tc_loss_body(pos_ref, neg_ref, out_ref):
    p = pos_ref[...]
    n = neg_ref[...]
    lp = jnp.minimum(p, 0.0) - jnp.log(1.0 + jnp.exp(-jnp.abs(p)))
    ln = jnp.minimum(-n, 0.0) - jnp.log(1.0 + jnp.exp(-jnp.abs(n)))
    out_ref[0, 0] = -(jnp.sum(lp) + jnp.sum(ln)) / B


_tc_loss = pl.pallas_call(
    _tc_loss_body,
    out_shape=jax.ShapeDtypeStruct((1, 1), jnp.float32),
    out_specs=pl.BlockSpec(memory_space=pltpu.SMEM),
)


def kernel(center_nodes, context_nodes, negative_nodes, u_weight, v_weight):
    center = center_nodes.astype(jnp.int32)
    context = context_nodes.astype(jnp.int32)
    neg_t = negative_nodes.astype(jnp.int32).T.reshape(K * B)
    pos, neg = _sc_scores(center, context, neg_t, u_weight, v_weight)
    pos2d = pos.reshape(B // 128, 128)
    neg2d = neg.reshape(K * B // 128, 128)
    out = _tc_loss(pos2d, neg2d)
    return out[0, 0]
